# Initial kernel scaffold; baseline (speedup 1.0000x reference)
import jax
import jax.numpy as jnp
from jax.experimental import pallas as pl


def kernel(attn_bias, spatial_pos, edge_input, attn_edge_type, edge_encoder_weight, spatial_pos_encoder_weight, graph_token_virtual_distance_weight):
    raise NotImplementedError("write your pallas kernel here")


# speedup vs baseline: 24.9574x; 24.9574x over previous
import functools

import jax
import jax.numpy as jnp
from jax import lax
from jax.experimental import pallas as pl
from jax.experimental.pallas import tpu as pltpu
from jax.experimental.pallas import tpu_sc as plsc

B = 16
N = 256
N1 = N + 1
H = 16
NUM_EDGES = 16384
NUM_SPATIAL = 512
EDGE_DIM = 4
NC, NS, L = 2, 16, 16
NW = NC * NS
TASKS = B * N
TPW = TASKS // NW


def _sc_bias(ctable, sp2, et2, *, interpret=False):
    mesh = plsc.VectorSubcoreMesh(
        core_axis_name="c", subcore_axis_name="s",
        num_cores=NC, num_subcores=NS)

    @functools.partial(
        pl.kernel,
        out_type=jax.ShapeDtypeStruct((B * H * N, N), jnp.float32),
        mesh=mesh,
        scratch_types=[
            pltpu.VMEM((N,), jnp.int32),
            pltpu.VMEM((N * EDGE_DIM,), jnp.int32),
            pltpu.VMEM((EDGE_DIM, N), jnp.int32),
            pltpu.VMEM((N, H), jnp.float32),
            pltpu.VMEM((H, N), jnp.float32),
            pltpu.VMEM((L,), jnp.int32),
            pltpu.SemaphoreType.DMA,
            pltpu.SemaphoreType.DMA,
        ],
        compiler_params=pltpu.CompilerParams(
            needs_layout_passes=False, use_tc_tiling_on_sc=False),
        interpret=interpret,
    )
    def k(ct_hbm, sp_hbm, et_hbm, out_hbm,
          sp_v, et_v, idx_ed, bias_v, trans_v, dst_v, gsem, ssem):
        wid = lax.axis_index("c") * NS + lax.axis_index("s")
        iota = lax.iota(jnp.int32, L)

        def task(t, _):
            r = wid * TPW + t
            pltpu.sync_copy(sp_hbm.at[r], sp_v)
            pltpu.sync_copy(et_hbm.at[r], et_v)
            for d in range(EDGE_DIM):
                for c in range(N // L):
                    vals = plsc.load_gather(
                        et_v, [iota * EDGE_DIM + (c * L * EDGE_DIM + d)])
                    idx_ed[d, pl.ds(c * L, L)] = vals + (NUM_SPATIAL + 1)
            pltpu.async_copy(ct_hbm.at[sp_v], bias_v, gsem).wait()
            cps = [pltpu.async_copy(ct_hbm.at[idx_ed.at[d]], bias_v, gsem,
                                    add=True)
                   for d in range(EDGE_DIM)]
            for cp in cps:
                cp.wait()
            for h in range(H):
                hvec = iota * 0 + h
                for c in range(N // L):
                    vals = plsc.load_gather(bias_v, [c * L + iota, hvec])
                    trans_v[h, pl.ds(c * L, L)] = vals
            b = r // N
            i = r % N
            dst_v[...] = iota * N + (b * (H * N) + i)
            pltpu.async_copy(trans_v, out_hbm.at[dst_v], ssem).wait()
            return ()

        lax.fori_loop(0, TPW, task, ())

    return k(ctable, sp2, et2)


def _asm_body(ab_ref, e_ref, t_ref, o_ref):
    h = pl.program_id(1)
    tv = t_ref[0, h]
    ab2 = ab_ref[0] * 2.0
    e = e_ref[0, 0]
    o_ref[0, 0, 0:1, :] = ab2[0:1, :] + tv
    o_ref[0, 0, 1:, 0:1] = ab2[1:, 0:1] + tv
    o_ref[0, 0, 1:, 1:] = ab2[1:, 1:] + e


def _tc_assemble(ab, et4, t, *, interpret=False):
    return pl.pallas_call(
        _asm_body,
        grid=(B, H),
        in_specs=[
            pl.BlockSpec((1, N1, N1), lambda b, h: (b, 0, 0)),
            pl.BlockSpec((1, 1, N, N), lambda b, h: (b, h, 0, 0)),
            pl.BlockSpec(memory_space=pltpu.SMEM),
        ],
        out_specs=pl.BlockSpec((1, 1, N1, N1), lambda b, h: (b, h, 0, 0)),
        out_shape=jax.ShapeDtypeStruct((B, H, N1, N1), jnp.float32),
        interpret=interpret,
    )(ab, et4, t)


def kernel(attn_bias, spatial_pos, edge_input, attn_edge_type,
           edge_encoder_weight, spatial_pos_encoder_weight,
           graph_token_virtual_distance_weight):
    sw0 = spatial_pos_encoder_weight.at[0].set(0.0)
    ew0 = edge_encoder_weight.at[0].set(0.0) * 0.25
    ctable = jnp.concatenate([sw0, ew0], axis=0)
    sp2 = spatial_pos.reshape(B * N, N)
    et2 = attn_edge_type.reshape(B * N, N * EDGE_DIM)
    et_flat = _sc_bias(ctable, sp2, et2)
    et4 = et_flat.reshape(B, H, N, N)
    return _tc_assemble(attn_bias, et4, graph_token_virtual_distance_weight)
